# Initial kernel scaffold; baseline (speedup 1.0000x reference)
import functools

import jax
import jax.numpy as jnp
from jax import lax
from jax.experimental import pallas as pl
from jax.experimental.pallas import tpu as pltpu
from jax.experimental.pallas import tpu_sc as plsc

N = 10000
E = 320000
D = 128
EPS = 1e-5

NC = 2
NS = 16
NW = NC * NS
EW = E // NW
C = 80
NCHUNK = EW // C
RPT = N // NS
RZ = 125
NRW = RPT // RZ


def _sc_agg_body(x_hbm, src_hbm, dst_hbm, out_hbm,
                 src_v, dst_v, rows_v, zrow_v, acc_sh, sem):
    c = lax.axis_index("c")
    s = lax.axis_index("s")
    wid = s * NC + c

    def zfill(i, carry):
        zrow_v[i // 8, pl.ds((i % 8) * 16, 16)] = jnp.zeros((16,), jnp.float32)
        return carry

    lax.fori_loop(0, RZ * 8, zfill, 0)
    for k in range(NRW):
        pltpu.sync_copy(zrow_v, acc_sh.at[pl.ds(s * RPT + k * RZ, RZ)])
    plsc.subcore_barrier()

    base = wid * EW

    def body(t, carry):
        off = base + t * C
        pltpu.sync_copy(src_hbm.at[pl.ds(off, C)], src_v)
        pltpu.sync_copy(dst_hbm.at[pl.ds(off, C)], dst_v)
        pltpu.async_copy(x_hbm.at[src_v], rows_v, sem).wait()
        pltpu.sync_copy(rows_v, acc_sh.at[dst_v], add=True)
        return carry

    lax.fori_loop(0, NCHUNK, body, 0)
    plsc.subcore_barrier()

    for k in range(NRW):
        r = s * RPT + k * RZ
        pltpu.sync_copy(acc_sh.at[pl.ds(r, RZ)], out_hbm.at[pl.ds(c * N + r, RZ)])


_sc_agg = functools.partial(
    pl.kernel,
    mesh=plsc.VectorSubcoreMesh(core_axis_name="c", subcore_axis_name="s"),
    out_type=jax.ShapeDtypeStruct((2 * N, D), jnp.float32),
    scratch_types=[
        pltpu.VMEM((C,), jnp.int32),
        pltpu.VMEM((C,), jnp.int32),
        pltpu.VMEM((C, D), jnp.float32),
        pltpu.VMEM((RZ, D), jnp.float32),
        pltpu.VMEM_SHARED((N, D), jnp.float32),
        pltpu.SemaphoreType.DMA,
    ],
)(_sc_agg_body)


def _tc_layer_body(p_ref, W_ref, b_ref, g_ref, be_ref, o_ref):
    a = p_ref[:N, :] + p_ref[N:, :]
    y = jnp.dot(a, W_ref[...], preferred_element_type=jnp.float32) + b_ref[...]
    mu = jnp.mean(y, axis=0, keepdims=True)
    d = y - mu
    var = jnp.mean(d * d, axis=0, keepdims=True)
    yn = d * lax.rsqrt(var + EPS) * g_ref[...] + be_ref[...]
    o_ref[...] = jnp.maximum(yn, 0.0)


def _tc_layer(parts, W, b, g, be):
    return pl.pallas_call(
        _tc_layer_body,
        out_shape=jax.ShapeDtypeStruct((N, D), jnp.float32),
    )(parts, W, b.reshape(1, D), g.reshape(1, D), be.reshape(1, D))


def kernel(x, edge_index, W1, b1, g1, be1, W2, b2, g2, be2):
    ei = edge_index.astype(jnp.int32)
    src, dst = ei[0], ei[1]
    p1 = _sc_agg(x, src, dst)
    h1 = _tc_layer(p1, W1, b1, g1, be1)
    p2 = _sc_agg(h1, src, dst)
    return _tc_layer(p2, W2, b2, g2, be2)


# speedup vs baseline: 5.2061x; 5.2061x over previous
import functools

import jax
import jax.numpy as jnp
from jax import lax
from jax.experimental import pallas as pl
from jax.experimental.pallas import tpu as pltpu
from jax.experimental.pallas import tpu_sc as plsc

N = 10000
E = 320000
D = 128
EPS = 1e-5

NC = 2
NS = 16
NW = NC * NS
EW = E // NW
C = 80
NCHUNK = EW // C
NP = 10240
RPT = NP // NS
RZ = 128
NRW = RPT // RZ


def _sc_agg_body(x_hbm, src_hbm, dst_hbm, out_hbm,
                 src_v, dst_v, rows_v, zrow_v, acc_sh, sem):
    c = lax.axis_index("c")
    s = lax.axis_index("s")
    wid = s * NC + c

    def zfill(i, carry):
        zrow_v[i // 8, pl.ds((i % 8) * 16, 16)] = jnp.zeros((16,), jnp.float32)
        return carry

    lax.fori_loop(0, RZ * 8, zfill, 0)
    for k in range(NRW):
        pltpu.sync_copy(zrow_v, acc_sh.at[pl.ds(s * RPT + k * RZ, RZ)])
    plsc.subcore_barrier()

    base = wid * EW

    def body(t, carry):
        off = base + t * C
        pltpu.sync_copy(src_hbm.at[pl.ds(off, C)], src_v)
        pltpu.sync_copy(dst_hbm.at[pl.ds(off, C)], dst_v)
        pltpu.async_copy(x_hbm.at[src_v], rows_v, sem).wait()
        pltpu.sync_copy(rows_v, acc_sh.at[dst_v], add=True)
        return carry

    lax.fori_loop(0, NCHUNK, body, 0)
    plsc.subcore_barrier()

    for k in range(NRW):
        r = s * RPT + k * RZ
        pltpu.sync_copy(acc_sh.at[pl.ds(r, RZ)], out_hbm.at[pl.ds(c * NP + r, RZ)])


@functools.lru_cache(maxsize=None)
def _get_sc_agg():
    return pl.kernel(
        _sc_agg_body,
        mesh=plsc.VectorSubcoreMesh(core_axis_name="c", subcore_axis_name="s"),
        out_type=jax.ShapeDtypeStruct((2 * NP, D), jnp.float32),
        scratch_types=[
            pltpu.VMEM((C,), jnp.int32),
            pltpu.VMEM((C,), jnp.int32),
            pltpu.VMEM((C, D), jnp.float32),
            pltpu.VMEM((RZ, D), jnp.float32),
            pltpu.VMEM_SHARED((NP, D), jnp.float32),
            pltpu.SemaphoreType.DMA,
        ],
    )


def _tc_layer_body(p_ref, W_ref, b_ref, g_ref, be_ref, o_ref):
    a = p_ref[:N, :] + p_ref[NP:NP + N, :]
    y = jnp.dot(a, W_ref[...], preferred_element_type=jnp.float32) + b_ref[...]
    mu = jnp.mean(y, axis=0, keepdims=True)
    d = y - mu
    var = jnp.mean(d * d, axis=0, keepdims=True)
    yn = d * lax.rsqrt(var + EPS) * g_ref[...] + be_ref[...]
    o_ref[...] = jnp.maximum(yn, 0.0)


def _tc_layer(parts, W, b, g, be):
    return pl.pallas_call(
        _tc_layer_body,
        out_shape=jax.ShapeDtypeStruct((N, D), jnp.float32),
    )(parts, W, b.reshape(1, D), g.reshape(1, D), be.reshape(1, D))


def kernel(x, edge_index, W1, b1, g1, be1, W2, b2, g2, be2):
    ei = edge_index.astype(jnp.int32)
    src, dst = ei[0], ei[1]
    sc_agg = _get_sc_agg()
    p1 = sc_agg(x, src, dst)
    h1 = _tc_layer(p1, W1, b1, g1, be1)
    p2 = sc_agg(h1, src, dst)
    return _tc_layer(p2, W2, b2, g2, be2)
